# baseline (speedup 1.0000x reference)
import functools

import jax
import jax.numpy as jnp
from jax import lax
from jax.experimental import pallas as pl
from jax.experimental.pallas import tpu as pltpu
from jax.experimental.pallas import tpu_sc as plsc

NTOKEN = 1000000
NINP = 32
BATCH = 16384
SEQ = 50

NC = 2
NS = 16
NW = NC * NS
BW = BATCH // NW
NBT = BW // 128
NG = NINP // 8
NCHUNK = SEQ * NBT


def _emb_body(idx_hbm, idxq_hbm, table_hbm, out_hbm, idx_v, idxq_v, a0, a1, b0, b1, gsems, wsems):
    wid = lax.axis_index("s") * NC + lax.axis_index("c")
    w0 = wid * NBT
    pltpu.sync_copy(idx_hbm.at[:, pl.ds(wid * BW, BW)], idx_v)
    pltpu.sync_copy(idxq_hbm.at[:, pl.ds(wid * BW, BW)], idxq_v)

    A = (a0, a1)
    B = (b0, b1)

    def fire_gather(c, p):
        l = c // NBT
        btl = c - l * NBT
        src = table_hbm.at[idxq_v.at[l, pl.ds(btl * 128, 128)]]
        pltpu.make_async_copy(src, A[p], gsems[p]).start()

    def wait_gather(c, p):
        l = c // NBT
        btl = c - l * NBT
        src = table_hbm.at[idxq_v.at[l, pl.ds(btl * 128, 128)]]
        pltpu.make_async_copy(src, A[p], gsems[p]).wait()

    def fire_wb(c, p):
        l = c // NBT
        btl = c - l * NBT
        for g in range(NG):
            pltpu.make_async_copy(B[p].at[g], out_hbm.at[l, g, w0 + btl], wsems[p]).start()

    def wait_wb(c, p):
        l = c // NBT
        btl = c - l * NBT
        for g in range(NG):
            pltpu.make_async_copy(B[p].at[g], out_hbm.at[l, g, w0 + btl], wsems[p]).wait()

    def transpose(c, p):
        l = c // NBT
        btl = c - l * NBT
        wv = []
        for k in range(8):
            v = idx_v[l, pl.ds(btl * 128 + k * 16, 16)]
            wv.append((v & 3) * 32)

        @plsc.parallel_loop(0, NINP, unroll=4)
        def _(f):
            g = f // 8
            fi = f - g * 8
            fvec = jnp.full((16,), f, jnp.int32)
            for k in range(8):
                bvec = jnp.arange(16, dtype=jnp.int32) + (k * 16)
                val = plsc.load_gather(A[p], [bvec, wv[k] + fvec])
                B[p][g, fi, pl.ds(k * 16, 16)] = val

    fire_gather(0, 0)

    def group(i, carry):
        c0 = 2 * i
        c1 = 2 * i + 1

        @pl.when(i > 0)
        def _():
            wait_wb(c1 - 2, 1)

        fire_gather(c1, 1)
        wait_gather(c0, 0)

        @pl.when(i > 0)
        def _():
            wait_wb(c0 - 2, 0)

        transpose(c0, 0)
        fire_wb(c0, 0)

        @pl.when(i < (NCHUNK // 2) - 1)
        def _():
            fire_gather(c0 + 2, 0)

        wait_gather(c1, 1)
        transpose(c1, 1)
        fire_wb(c1, 1)
        return carry

    lax.fori_loop(0, NCHUNK // 2, group, 0)
    wait_wb(NCHUNK - 2, 0)
    wait_wb(NCHUNK - 1, 1)


@jax.jit
def _emb(idxt, idxq, table):
    mesh = plsc.VectorSubcoreMesh(core_axis_name="c", subcore_axis_name="s")
    k = pl.kernel(
        _emb_body,
        mesh=mesh,
        compiler_params=pltpu.CompilerParams(
            use_tc_tiling_on_sc=False, needs_layout_passes=False
        ),
        out_type=jax.ShapeDtypeStruct((SEQ, NG, BATCH // 128, 8, 128), jnp.float32),
        scratch_types=[
            pltpu.VMEM((SEQ, BW), jnp.int32),
            pltpu.VMEM((SEQ, BW), jnp.int32),
            pltpu.VMEM((128, 128), jnp.float32),
            pltpu.VMEM((128, 128), jnp.float32),
            pltpu.VMEM((NG, 8, 128), jnp.float32),
            pltpu.VMEM((NG, 8, 128), jnp.float32),
            [pltpu.SemaphoreType.DMA] * 2,
            [pltpu.SemaphoreType.DMA] * 2,
        ],
    )
    return k(idxt, idxq, table)


def kernel(input_ids, embed_weight):
    idxt = input_ids.T.astype(jnp.int32)
    idxq = lax.shift_right_logical(idxt, 2)
    tabq = embed_weight.reshape(NTOKEN // 4, 128)
    o5 = _emb(idxt, idxq, tabq)
    return o5.transpose(2, 4, 0, 1, 3).reshape(BATCH, SEQ, NINP)
